# baseline (speedup 1.0000x reference)
import functools

import jax
import jax.numpy as jnp
from jax import lax
from jax.experimental import pallas as pl
from jax.experimental.pallas import tpu as pltpu
from jax.experimental.pallas import tpu_sc as plsc

_BS = 4096
_B = 256


def _sc_body(ts_hbm, tab_hbm, out_hbm, idx_v, a_v, sa_v, sb_v, sem):
    wid = lax.axis_index("s") * 2 + lax.axis_index("c")

    @pl.when(wid < _B // 16)
    def _():
        base = wid * 16
        pltpu.sync_copy(ts_hbm.at[pl.ds(base, 16)], idx_v)
        pltpu.async_copy(tab_hbm.at[idx_v], a_v, sem).wait()
        a = a_v[...]

        def _sqrt(v):
            half = 0.5 * v
            y = lax.bitcast_convert_type(
                jnp.full((16,), 0x5F3759DF, jnp.int32)
                - lax.shift_right_logical(
                    lax.bitcast_convert_type(v, jnp.int32), 1
                ),
                jnp.float32,
            )
            for _ in range(3):
                y = y * (1.5 - half * y * y)
            return v * y

        sa_v[...] = _sqrt(a)
        sb_v[...] = _sqrt(1.0 - a)
        pltpu.sync_copy(sa_v, out_hbm.at[0, pl.ds(base, 16)])
        pltpu.sync_copy(sb_v, out_hbm.at[1, pl.ds(base, 16)])


@functools.partial(
    pl.kernel,
    mesh=plsc.VectorSubcoreMesh(core_axis_name="c", subcore_axis_name="s"),
    out_type=jax.ShapeDtypeStruct((2, _B), jnp.float32),
    scratch_types=[
        pltpu.VMEM((16,), jnp.int32),
        pltpu.VMEM((16,), jnp.float32),
        pltpu.VMEM((16,), jnp.float32),
        pltpu.VMEM((16,), jnp.float32),
        pltpu.SemaphoreType.DMA,
    ],
)
def _sc_scales(ts_hbm, tab_hbm, out_hbm, idx_v, a_v, sa_v, sb_v, sem):
    _sc_body(ts_hbm, tab_hbm, out_hbm, idx_v, a_v, sa_v, sb_v, sem)


def _tc_body(s_ref, x_ref, n_ref, o_ref):
    o_ref[...] = s_ref[0:1, :] * x_ref[...] + s_ref[1:2, :] * n_ref[...]


def kernel(original_samples, noise, timesteps, alphas_cumprod):
    b, c, h, w = original_samples.shape
    p = c * h * w
    xt = original_samples.transpose(1, 2, 3, 0).reshape(p, b)
    nt = noise.transpose(1, 2, 3, 0).reshape(p, b)
    ts = timesteps.astype(jnp.int32)
    tab = jnp.pad(alphas_cumprod, (0, 1024 - alphas_cumprod.shape[0]))

    scales = _sc_scales(ts, tab)

    blk = pl.BlockSpec((_BS, b), lambda i: (i, 0))
    out = pl.pallas_call(
        _tc_body,
        grid=(p // _BS,),
        in_specs=[pl.BlockSpec((2, b), lambda i: (0, 0)), blk, blk],
        out_specs=blk,
        out_shape=jax.ShapeDtypeStruct((p, b), jnp.float32),
        compiler_params=pltpu.CompilerParams(dimension_semantics=("arbitrary",)),
    )(scales, xt, nt)
    return out.reshape(c, h, w, b).transpose(3, 0, 1, 2)
